# Initial kernel scaffold; baseline (speedup 1.0000x reference)
import jax
import jax.numpy as jnp
from jax.experimental import pallas as pl


def kernel(x, edge2, edge2_r, ei2, params):
    raise NotImplementedError("write your pallas kernel here")


# speedup vs baseline: 17.3194x; 17.3194x over previous
import functools

import jax
import jax.numpy as jnp
from jax import lax
from jax.experimental import pallas as pl
from jax.experimental.pallas import tpu as pltpu
from jax.experimental.pallas import tpu_sc as plsc

_N = 10000
_E = 320000
_F = 128
_L = 32
_NCLS = 100
_EPS = 1e-5

_NS = 16
_EPT = _E // _NS
_DEG_PAD = 10240
_DCH = 2000
_C = 80
_RPT = _N // _NS

_mesh = plsc.VectorSubcoreMesh(core_axis_name="c", subcore_axis_name="s")
_sc_params = pltpu.CompilerParams(needs_layout_passes=False,
                                  use_tc_tiling_on_sc=False)


def _deg_body(dst_f, dst_r, out_f, out_r, idx_v, deg_v):
    cid = lax.axis_index("c")
    sid = lax.axis_index("s")
    zero16 = jnp.zeros((16,), jnp.float32)
    one16 = jnp.ones((16,), jnp.float32)

    def zbody(i, c):
        deg_v[pl.ds(i * 16, 16)] = zero16
        return c

    lax.fori_loop(0, _DEG_PAD // 16, zbody, 0)

    def run(dst_hbm, out_hbm):
        ebase = sid * _EPT

        def chunk(ci, c):
            pltpu.sync_copy(dst_hbm.at[pl.ds(ebase + ci * _DCH, _DCH)], idx_v)

            def inner(i, c2):
                idx = idx_v[pl.ds(i * 16, 16)]
                plsc.addupdate_scatter(deg_v, [idx], one16)
                return c2

            return lax.fori_loop(0, _DCH // 16, inner, c)

        lax.fori_loop(0, _EPT // _DCH, chunk, 0)
        pltpu.sync_copy(deg_v, out_hbm.at[sid])

    @pl.when(cid == 0)
    def _():
        run(dst_f, out_f)

    @pl.when(cid == 1)
    def _():
        run(dst_r, out_r)


_deg_call = functools.partial(
    pl.kernel,
    out_type=(
        jax.ShapeDtypeStruct((_NS, _DEG_PAD), jnp.float32),
        jax.ShapeDtypeStruct((_NS, _DEG_PAD), jnp.float32),
    ),
    mesh=_mesh,
    compiler_params=_sc_params,
    scratch_types=[
        pltpu.VMEM((_DCH,), jnp.int32),
        pltpu.VMEM((_DEG_PAD,), jnp.float32),
    ],
)(_deg_body)


def _agg_body(g_f, g_r, src_f, src_r, dst_f, dst_r, zeros,
              out_f, out_r, si_v, di_v, rows_v, sem, acc_sh):
    cid = lax.axis_index("c")
    sid = lax.axis_index("s")
    r0 = sid * _RPT

    def run(g_hbm, s_hbm, d_hbm, out_hbm):
        pltpu.sync_copy(zeros.at[pl.ds(r0, _RPT)], acc_sh.at[pl.ds(r0, _RPT)])
        plsc.subcore_barrier()
        ebase = sid * _EPT

        def chunk(ci, c):
            b = ebase + ci * _C
            pltpu.sync_copy(s_hbm.at[pl.ds(b, _C)], si_v)
            pltpu.sync_copy(d_hbm.at[pl.ds(b, _C)], di_v)
            pltpu.async_copy(g_hbm.at[si_v], rows_v, sem).wait()
            pltpu.sync_copy(rows_v, acc_sh.at[di_v], add=True)
            return c

        lax.fori_loop(0, _EPT // _C, chunk, 0)
        plsc.subcore_barrier()
        pltpu.sync_copy(acc_sh.at[pl.ds(r0, _RPT)], out_hbm.at[pl.ds(r0, _RPT)])

    @pl.when(cid == 0)
    def _():
        run(g_f, src_f, dst_f, out_f)

    @pl.when(cid == 1)
    def _():
        run(g_r, src_r, dst_r, out_r)


_agg_call = functools.partial(
    pl.kernel,
    out_type=(
        jax.ShapeDtypeStruct((_N, _L), jnp.float32),
        jax.ShapeDtypeStruct((_N, _L), jnp.float32),
    ),
    mesh=_mesh,
    compiler_params=_sc_params,
    scratch_types=[
        pltpu.VMEM((_C,), jnp.int32),
        pltpu.VMEM((_C,), jnp.int32),
        pltpu.VMEM((_C, _L), jnp.float32),
        pltpu.SemaphoreType.DMA,
        pltpu.VMEM_SHARED((_N, _L), jnp.float32),
    ],
)(_agg_body)


def _graph_norm(h, gw, gb, gms):
    mean = jnp.mean(h, axis=0, keepdims=True)
    o = h - gms * mean
    var = jnp.mean(o * o, axis=0, keepdims=True)
    return gw * o * lax.rsqrt(var + _EPS) + gb


_tc_params = pltpu.CompilerParams(vmem_limit_bytes=100 * 1024 * 1024)


def _tc1_body(x_ref, w1_ref, w1r_ref, dv_ref, g1_ref, g1r_ref):
    x = x_ref[...]
    g1_ref[...] = jnp.dot(x, w1_ref[...],
                          preferred_element_type=jnp.float32) * dv_ref[:, 0:1]
    g1r_ref[...] = jnp.dot(x, w1r_ref[...],
                           preferred_element_type=jnp.float32) * dv_ref[:, 1:2]


_tc1 = pl.pallas_call(
    _tc1_body,
    out_shape=(
        jax.ShapeDtypeStruct((_N, _L), jnp.float32),
        jax.ShapeDtypeStruct((_N, _L), jnp.float32),
    ),
    compiler_params=_tc_params,
)


def _tc2_body(a1_ref, a1r_ref, g1_ref, g1r_ref, dv_ref, p_ref,
              pr_ref, g2_ref, g2r_ref):
    def branch(a_ref, g_ref, dv, prm):
        b, gw, gb, gms = (prm[0:1], prm[1:2], prm[2:3], prm[3:4])
        c = dv * (a_ref[...] + g_ref[...]) + b
        o = _graph_norm(c, gw, gb, gms)
        mu = jnp.mean(o, axis=1, keepdims=True)
        var = jnp.mean((o - mu) ** 2, axis=1, keepdims=True)
        o = (o - mu) * lax.rsqrt(var + _EPS)
        return jnp.maximum(o, 0.0)

    x1 = (branch(a1_ref, g1_ref, dv_ref[:, 0:1], p_ref[...])
          + branch(a1r_ref, g1r_ref, dv_ref[:, 1:2], pr_ref[...]))
    g2_ref[...] = x1 * dv_ref[:, 0:1]
    g2r_ref[...] = x1 * dv_ref[:, 1:2]


_tc2 = pl.pallas_call(
    _tc2_body,
    out_shape=(
        jax.ShapeDtypeStruct((_N, _L), jnp.float32),
        jax.ShapeDtypeStruct((_N, _L), jnp.float32),
    ),
    compiler_params=_tc_params,
)


def _tc3_body(a2_ref, a2r_ref, g2_ref, g2r_ref, dv_ref,
              w2_ref, w2r_ref, p_ref, pr_ref, y_ref):
    def branch(a_ref, g_ref, dv, w_ref, prm):
        b, gw, gb, gms = (prm[0:1], prm[1:2], prm[2:3], prm[3:4])
        u = dv * (a_ref[...] + g_ref[...])
        h = jnp.dot(u, w_ref[...], preferred_element_type=jnp.float32) + b
        return _graph_norm(h, gw, gb, gms)

    s = (branch(a2_ref, g2_ref, dv_ref[:, 0:1], w2_ref, p_ref[...])
         + branch(a2r_ref, g2r_ref, dv_ref[:, 1:2], w2r_ref, pr_ref[...]))
    y_ref[...] = jax.nn.sigmoid(s[: _N // 2])


_tc3 = pl.pallas_call(
    _tc3_body,
    out_shape=jax.ShapeDtypeStruct((_N // 2, _NCLS), jnp.float32),
    compiler_params=_tc_params,
)


def kernel(x, edge2, edge2_r, ei2, params):
    del ei2
    src_f, dst_f = edge2[0], edge2[1]
    src_r, dst_r = edge2_r[0], edge2_r[1]

    degp_f, degp_r = _deg_call(dst_f, dst_r)
    dinv_f = lax.rsqrt(degp_f.sum(axis=0)[:_N] + 1.0)
    dinv_r = lax.rsqrt(degp_r.sum(axis=0)[:_N] + 1.0)
    dv = jnp.stack([dinv_f, dinv_r], axis=1)

    g1, g1r = _tc1(x, params["W1"], params["W1_r"], dv)

    zeros = jnp.zeros((_N, _L), jnp.float32)
    a1, a1r = _agg_call(g1, g1r, src_f, src_r, dst_f, dst_r, zeros)

    p1 = jnp.stack([params["b1"], params["gw1"], params["gb1"], params["gms1"]])
    p1r = jnp.stack([params["b1_r"], params["gw1_r"], params["gb1_r"],
                     params["gms1_r"]])
    g2, g2r = _tc2(a1, a1r, g1, g1r, dv, p1, p1r)

    a2, a2r = _agg_call(g2, g2r, src_f, src_r, dst_f, dst_r, zeros)

    p2 = jnp.stack([params["b2"], params["gw2"], params["gb2"], params["gms2"]])
    p2r = jnp.stack([params["b2_r"], params["gw2_r"], params["gb2_r"],
                     params["gms2_r"]])
    return _tc3(a2, a2r, g2, g2r, dv,
                params["W2"], params["W2_r"], p2, p2r)


# speedup vs baseline: 58.2175x; 3.3614x over previous
import functools

import jax
import jax.numpy as jnp
from jax import lax
from jax.experimental import pallas as pl
from jax.experimental.pallas import tpu as pltpu
from jax.experimental.pallas import tpu_sc as plsc

_N = 10000
_E = 320000
_F = 128
_L = 32
_NCLS = 100
_EPS = 1e-5

_NS = 16
_EPT = _E // _NS
_DEG_PAD = 10240
_DCH = 2000
_RPT = _N // _NS
_CW = 80
_BR = 10
_NBLK = _EPT // (_BR * _CW)
_ROWS_PT = _EPT // _CW

_mesh = plsc.VectorSubcoreMesh(core_axis_name="c", subcore_axis_name="s")
_sc_params = pltpu.CompilerParams(needs_layout_passes=False,
                                  use_tc_tiling_on_sc=False)


def _deg_body(dst_f, dst_r, out_f, out_r, idx_v, deg_v):
    cid = lax.axis_index("c")
    sid = lax.axis_index("s")
    zero16 = jnp.zeros((16,), jnp.float32)
    one16 = jnp.ones((16,), jnp.float32)

    def zbody(i, c):
        deg_v[pl.ds(i * 16, 16)] = zero16
        return c

    lax.fori_loop(0, _DEG_PAD // 16, zbody, 0)

    def run(dst_hbm, out_hbm):
        rbase = sid * _ROWS_PT
        rows_per_chunk = _DCH // _CW

        def chunk(ci, c):
            pltpu.sync_copy(
                dst_hbm.at[pl.ds(rbase + ci * rows_per_chunk, rows_per_chunk)],
                idx_v)

            def inner(i, c2):
                idx = idx_v[i // (_CW // 16), pl.ds((i % (_CW // 16)) * 16, 16)]
                plsc.addupdate_scatter(deg_v, [idx], one16)
                return c2

            return lax.fori_loop(0, _DCH // 16, inner, c)

        lax.fori_loop(0, _EPT // _DCH, chunk, 0)
        pltpu.sync_copy(deg_v, out_hbm.at[sid])

    @pl.when(cid == 0)
    def _():
        run(dst_f, out_f)

    @pl.when(cid == 1)
    def _():
        run(dst_r, out_r)


_deg_call = functools.partial(
    pl.kernel,
    out_type=(
        jax.ShapeDtypeStruct((_NS, _DEG_PAD), jnp.float32),
        jax.ShapeDtypeStruct((_NS, _DEG_PAD), jnp.float32),
    ),
    mesh=_mesh,
    compiler_params=_sc_params,
    scratch_types=[
        pltpu.VMEM((_DCH // _CW, _CW), jnp.int32),
        pltpu.VMEM((_DEG_PAD,), jnp.float32),
    ],
)(_deg_body)


def _agg_body(g_f, g_r, src_f, src_r, dst_f, dst_r, zeros,
              out_f, out_r, sidx_v, didx_v, rows_v, semi, semg, sems, acc_sh):
    cid = lax.axis_index("c")
    sid = lax.axis_index("s")
    r0 = sid * _RPT

    def run(g_hbm, s_hbm, d_hbm, out_hbm):
        pltpu.sync_copy(zeros.at[pl.ds(r0, _RPT)], acc_sh.at[pl.ds(r0, _RPT)])
        plsc.subcore_barrier()
        row0 = sid * _ROWS_PT

        def idx_start(t, buf):
            pltpu.async_copy(s_hbm.at[pl.ds(row0 + t * _BR, _BR)],
                             sidx_v.at[buf], semi)
            pltpu.async_copy(d_hbm.at[pl.ds(row0 + t * _BR, _BR)],
                             didx_v.at[buf], semi)

        def idx_wait(buf):
            pltpu.make_async_copy(s_hbm.at[pl.ds(row0, _BR)],
                                  sidx_v.at[buf], semi).wait()
            pltpu.make_async_copy(d_hbm.at[pl.ds(row0, _BR)],
                                  didx_v.at[buf], semi).wait()

        def gat_start(tb):
            for r in range(_BR):
                pltpu.async_copy(g_hbm.at[sidx_v.at[tb, r]],
                                 rows_v.at[tb, r], semg)

        def gat_wait(tb):
            for r in range(_BR):
                pltpu.make_async_copy(g_hbm.at[sidx_v.at[tb, r]],
                                      rows_v.at[tb, r], semg).wait()

        def scat_start(tb):
            for r in range(_BR):
                pltpu.async_copy(rows_v.at[tb, r],
                                 acc_sh.at[didx_v.at[tb, r]], sems, add=True)

        def scat_wait(buf):
            for r in range(_BR):
                pltpu.make_async_copy(rows_v.at[buf, r],
                                      acc_sh.at[didx_v.at[buf, r]],
                                      sems).wait()

        idx_start(0, 0)

        def body(t, carry):
            tb = t % 2
            idx_wait(tb)
            gat_start(tb)
            gat_wait(tb)

            @pl.when(t >= 1)
            def _():
                scat_wait(1 - tb)

            @pl.when(t + 1 < _NBLK)
            def _():
                idx_start(t + 1, 1 - tb)

            scat_start(tb)
            return carry

        lax.fori_loop(0, _NBLK, body, 0)
        scat_wait((_NBLK - 1) % 2)
        plsc.subcore_barrier()
        pltpu.sync_copy(acc_sh.at[pl.ds(r0, _RPT)], out_hbm.at[pl.ds(r0, _RPT)])

    @pl.when(cid == 0)
    def _():
        run(g_f, src_f, dst_f, out_f)

    @pl.when(cid == 1)
    def _():
        run(g_r, src_r, dst_r, out_r)


_agg_call = functools.partial(
    pl.kernel,
    out_type=(
        jax.ShapeDtypeStruct((_N, _L), jnp.float32),
        jax.ShapeDtypeStruct((_N, _L), jnp.float32),
    ),
    mesh=_mesh,
    compiler_params=_sc_params,
    scratch_types=[
        pltpu.VMEM((2, _BR, _CW), jnp.int32),
        pltpu.VMEM((2, _BR, _CW), jnp.int32),
        pltpu.VMEM((2, _BR, _CW, _L), jnp.float32),
        pltpu.SemaphoreType.DMA,
        pltpu.SemaphoreType.DMA,
        pltpu.SemaphoreType.DMA,
        pltpu.VMEM_SHARED((_N, _L), jnp.float32),
    ],
)(_agg_body)


def _graph_norm(h, gw, gb, gms):
    mean = jnp.mean(h, axis=0, keepdims=True)
    o = h - gms * mean
    var = jnp.mean(o * o, axis=0, keepdims=True)
    return gw * o * lax.rsqrt(var + _EPS) + gb


_tc_params = pltpu.CompilerParams(vmem_limit_bytes=100 * 1024 * 1024)


def _tc1_body(x_ref, w1_ref, w1r_ref, dv_ref, g1_ref, g1r_ref):
    x = x_ref[...]
    g1_ref[...] = jnp.dot(x, w1_ref[...],
                          preferred_element_type=jnp.float32) * dv_ref[:, 0:1]
    g1r_ref[...] = jnp.dot(x, w1r_ref[...],
                           preferred_element_type=jnp.float32) * dv_ref[:, 1:2]


_tc1 = pl.pallas_call(
    _tc1_body,
    out_shape=(
        jax.ShapeDtypeStruct((_N, _L), jnp.float32),
        jax.ShapeDtypeStruct((_N, _L), jnp.float32),
    ),
    compiler_params=_tc_params,
)


def _tc2_body(a1_ref, a1r_ref, g1_ref, g1r_ref, dv_ref, p_ref,
              pr_ref, g2_ref, g2r_ref):
    def branch(a_ref, g_ref, dv, prm):
        b, gw, gb, gms = (prm[0:1], prm[1:2], prm[2:3], prm[3:4])
        c = dv * (a_ref[...] + g_ref[...]) + b
        o = _graph_norm(c, gw, gb, gms)
        mu = jnp.mean(o, axis=1, keepdims=True)
        var = jnp.mean((o - mu) ** 2, axis=1, keepdims=True)
        o = (o - mu) * lax.rsqrt(var + _EPS)
        return jnp.maximum(o, 0.0)

    x1 = (branch(a1_ref, g1_ref, dv_ref[:, 0:1], p_ref[...])
          + branch(a1r_ref, g1r_ref, dv_ref[:, 1:2], pr_ref[...]))
    g2_ref[...] = x1 * dv_ref[:, 0:1]
    g2r_ref[...] = x1 * dv_ref[:, 1:2]


_tc2 = pl.pallas_call(
    _tc2_body,
    out_shape=(
        jax.ShapeDtypeStruct((_N, _L), jnp.float32),
        jax.ShapeDtypeStruct((_N, _L), jnp.float32),
    ),
    compiler_params=_tc_params,
)


def _tc3_body(a2_ref, a2r_ref, g2_ref, g2r_ref, dv_ref,
              w2_ref, w2r_ref, p_ref, pr_ref, y_ref):
    def branch(a_ref, g_ref, dv, w_ref, prm):
        b, gw, gb, gms = (prm[0:1], prm[1:2], prm[2:3], prm[3:4])
        u = dv * (a_ref[...] + g_ref[...])
        h = jnp.dot(u, w_ref[...], preferred_element_type=jnp.float32) + b
        return _graph_norm(h, gw, gb, gms)

    s = (branch(a2_ref, g2_ref, dv_ref[:, 0:1], w2_ref, p_ref[...])
         + branch(a2r_ref, g2r_ref, dv_ref[:, 1:2], w2r_ref, pr_ref[...]))
    y_ref[...] = jax.nn.sigmoid(s[: _N // 2])


_tc3 = pl.pallas_call(
    _tc3_body,
    out_shape=jax.ShapeDtypeStruct((_N // 2, _NCLS), jnp.float32),
    compiler_params=_tc_params,
)


def kernel(x, edge2, edge2_r, ei2, params):
    del ei2
    src_f, dst_f = edge2[0].reshape(-1, _CW), edge2[1].reshape(-1, _CW)
    src_r, dst_r = edge2_r[0].reshape(-1, _CW), edge2_r[1].reshape(-1, _CW)

    degp_f, degp_r = _deg_call(dst_f, dst_r)
    dinv_f = lax.rsqrt(degp_f.sum(axis=0)[:_N] + 1.0)
    dinv_r = lax.rsqrt(degp_r.sum(axis=0)[:_N] + 1.0)
    dv = jnp.stack([dinv_f, dinv_r], axis=1)

    g1, g1r = _tc1(x, params["W1"], params["W1_r"], dv)

    zeros = jnp.zeros((_N, _L), jnp.float32)
    a1, a1r = _agg_call(g1, g1r, src_f, src_r, dst_f, dst_r, zeros)

    p1 = jnp.stack([params["b1"], params["gw1"], params["gb1"], params["gms1"]])
    p1r = jnp.stack([params["b1_r"], params["gw1_r"], params["gb1_r"],
                     params["gms1_r"]])
    g2, g2r = _tc2(a1, a1r, g1, g1r, dv, p1, p1r)

    a2, a2r = _agg_call(g2, g2r, src_f, src_r, dst_f, dst_r, zeros)

    p2 = jnp.stack([params["b2"], params["gw2"], params["gb2"], params["gms2"]])
    p2r = jnp.stack([params["b2_r"], params["gw2_r"], params["gb2_r"],
                     params["gms2_r"]])
    return _tc3(a2, a2r, g2, g2r, dv,
                params["W2"], params["W2_r"], p2, p2r)
